# baseline (speedup 1.0000x reference)
import functools

import jax
import jax.numpy as jnp
from jax import lax
from jax.experimental import pallas as pl
from jax.experimental.pallas import tpu as pltpu
from jax.experimental.pallas import tpu_sc as plsc

_DIM = 4096
_NE = 64
_TOPK = 8
_T = 16384
_BT = 512
_NCHUNK = 2

_NC = 2
_NS = 16
_NW = _NC * _NS


def _matmul_body(x_ref, w_ref, o_ref):
    o_ref[...] = lax.dot_general(
        x_ref[...], w_ref[...],
        dimension_numbers=(((1,), (1,)), ((), ())),
        preferred_element_type=jnp.float32,
    )


def _scores_tc(x, weight):
    rows = x.shape[0]
    return pl.pallas_call(
        _matmul_body,
        grid=(rows // _BT,),
        in_specs=[
            pl.BlockSpec((_BT, _DIM), lambda i: (i, 0)),
            pl.BlockSpec((_NE, _DIM), lambda i: (0, 0)),
        ],
        out_specs=pl.BlockSpec((_BT, _NE), lambda i: (i, 0)),
        out_shape=jax.ShapeDtypeStruct((rows, _NE), jnp.float32),
    )(x, weight)


@functools.cache
def _topk_sc(rows):
    nrow = rows // _NW

    def _topk_body(scores_hbm, wout_hbm, iout_hbm, sbuf, wbuf, ibuf):
        wid = lax.axis_index("s") * _NC + lax.axis_index("c")
        base = wid * nrow
        pltpu.sync_copy(scores_hbm.at[pl.ds(base, nrow)], sbuf)

        lanes = lax.iota(jnp.int32, 16)
        in_lo = lanes < 8

        def _merge(ka, va, kb, vb):
            kb_r = lax.rev(kb, (0,))
            vb_r = lax.rev(vb, (0,))
            k = jnp.where(in_lo, ka, kb_r)
            v = jnp.where(in_lo, va, vb_r)
            return plsc.sort_key_val(k, v, descending=True)

        @plsc.parallel_loop(0, nrow, step=1, unroll=8)
        def _row(r):
            srt = []
            for j in range(4):
                k = sbuf[r, pl.ds(16 * j, 16)]
                srt.append(
                    plsc.sort_key_val(k, lanes + 16 * j, descending=True))
            k01, v01 = _merge(*srt[0], *srt[1])
            k23, v23 = _merge(*srt[2], *srt[3])
            kf, vf = _merge(k01, v01, k23, v23)
            m = jnp.max(kf)
            e = jnp.exp(kf - m)
            e = jnp.where(in_lo, e, 0.0)
            s = jnp.broadcast_to(jnp.sum(e), (16,))
            wbuf[r] = e / s
            ibuf[r] = vf

        pltpu.sync_copy(wbuf, wout_hbm.at[pl.ds(base, nrow)])
        pltpu.sync_copy(ibuf, iout_hbm.at[pl.ds(base, nrow)])

    return pl.kernel(
        _topk_body,
        out_type=(
            jax.ShapeDtypeStruct((rows, 16), jnp.float32),
            jax.ShapeDtypeStruct((rows, 16), jnp.int32),
        ),
        mesh=plsc.VectorSubcoreMesh(core_axis_name="c", subcore_axis_name="s"),
        compiler_params=pltpu.CompilerParams(
            needs_layout_passes=False, use_tc_tiling_on_sc=False),
        scratch_types=[
            pltpu.VMEM((nrow, _NE), jnp.float32),
            pltpu.VMEM((nrow, 16), jnp.float32),
            pltpu.VMEM((nrow, 16), jnp.int32),
        ],
    )


def kernel(x, weight):
    crows = _T // _NCHUNK
    sc = _topk_sc(crows)
    outs = [sc(_scores_tc(x[c * crows:(c + 1) * crows], weight))
            for c in range(_NCHUNK)]
    w16 = jnp.concatenate([o[0] for o in outs], axis=0)
    i16 = jnp.concatenate([o[1] for o in outs], axis=0)
    return (w16[:, :_TOPK], i16[:, :_TOPK])


# speedup vs baseline: 2.2223x; 2.2223x over previous
import functools

import jax
import jax.numpy as jnp
from jax import lax
from jax.experimental import pallas as pl
from jax.experimental.pallas import tpu as pltpu
from jax.experimental.pallas import tpu_sc as plsc

_DIM = 4096
_NE = 64
_TOPK = 8
_T = 16384
_BT = 512
_NCHUNK = 2

_NC = 2
_NS = 16
_NW = _NC * _NS


def _matmul_body(x_ref, w_ref, o_ref):
    o_ref[...] = lax.dot_general(
        x_ref[...], w_ref[...],
        dimension_numbers=(((1,), (1,)), ((), ())),
        preferred_element_type=jnp.float32,
    )


def _scores_tc(x, weight, rows, row0):
    blk0 = row0 // _BT
    return pl.pallas_call(
        _matmul_body,
        grid=(rows // _BT,),
        in_specs=[
            pl.BlockSpec((_BT, _DIM), lambda i: (blk0 + i, 0)),
            pl.BlockSpec((_NE, _DIM), lambda i: (0, 0)),
        ],
        out_specs=pl.BlockSpec((_BT, _NE), lambda i: (i, 0)),
        out_shape=jax.ShapeDtypeStruct((rows, _NE), jnp.float32),
    )(x, weight)


@functools.cache
def _topk_sc(rows):
    nrow = rows // _NW

    def _topk_body(scores_hbm, wout_hbm, iout_hbm, sbuf, wbuf, ibuf):
        wid = lax.axis_index("s") * _NC + lax.axis_index("c")
        base = wid * nrow
        pltpu.sync_copy(scores_hbm.at[pl.ds(base, nrow)], sbuf)

        lanes = lax.iota(jnp.int32, 16)
        in_lo = lanes < 8

        def _merge(ka, va, kb, vb):
            kb_r = lax.rev(kb, (0,))
            vb_r = lax.rev(vb, (0,))
            k = jnp.where(in_lo, ka, kb_r)
            v = jnp.where(in_lo, va, vb_r)
            return plsc.sort_key_val(k, v, descending=True)

        @plsc.parallel_loop(0, nrow, step=1, unroll=8)
        def _row(r):
            srt = []
            for j in range(4):
                k = sbuf[r, pl.ds(16 * j, 16)]
                srt.append(
                    plsc.sort_key_val(k, lanes + 16 * j, descending=True))
            k01, v01 = _merge(*srt[0], *srt[1])
            k23, v23 = _merge(*srt[2], *srt[3])
            kf, vf = _merge(k01, v01, k23, v23)
            m = jnp.max(kf)
            e = jnp.exp(kf - m)
            e = jnp.where(in_lo, e, 0.0)
            s = jnp.broadcast_to(jnp.sum(e), (16,))
            wbuf[r] = e / s
            ibuf[r] = vf

        pltpu.sync_copy(wbuf, wout_hbm.at[pl.ds(base, nrow)])
        pltpu.sync_copy(ibuf, iout_hbm.at[pl.ds(base, nrow)])

    return pl.kernel(
        _topk_body,
        out_type=(
            jax.ShapeDtypeStruct((rows, 16), jnp.float32),
            jax.ShapeDtypeStruct((rows, 16), jnp.int32),
        ),
        mesh=plsc.VectorSubcoreMesh(core_axis_name="c", subcore_axis_name="s"),
        compiler_params=pltpu.CompilerParams(
            needs_layout_passes=False, use_tc_tiling_on_sc=False),
        scratch_types=[
            pltpu.VMEM((nrow, _NE), jnp.float32),
            pltpu.VMEM((nrow, 16), jnp.float32),
            pltpu.VMEM((nrow, 16), jnp.int32),
        ],
    )


def kernel(x, weight):
    crows = _T // _NCHUNK
    sc = _topk_sc(crows)
    outs = [sc(_scores_tc(x, weight, crows, c * crows))
            for c in range(_NCHUNK)]
    w16 = jnp.concatenate([o[0] for o in outs], axis=0)
    i16 = jnp.concatenate([o[1] for o in outs], axis=0)
    return (w16[:, :_TOPK], i16[:, :_TOPK])
